# baseline (speedup 1.0000x reference)
import functools

import jax
import jax.numpy as jnp
from jax import lax
from jax.experimental import pallas as pl
from jax.experimental.pallas import tpu as pltpu
from jax.experimental.pallas import tpu_sc as plsc

_V = 1000000
_D = 64
_NW = 32
_FULL_BLOCKS = _V // 128
_TAIL = _V - _FULL_BLOCKS * 128
_BPW = _FULL_BLOCKS // _NW
_EXTRA = _FULL_BLOCKS % _NW


def _transpose_call(wt, tail1d):
    mesh = plsc.VectorSubcoreMesh(core_axis_name="c", subcore_axis_name="s")

    @functools.partial(
        pl.kernel,
        out_type=jax.ShapeDtypeStruct((_V * _D,), jnp.float32),
        mesh=mesh,
        scratch_types=[
            pltpu.VMEM((64, 128), jnp.float32),
            pltpu.VMEM((64, 128), jnp.float32),
            pltpu.VMEM((8192,), jnp.float32),
            pltpu.VMEM((8192,), jnp.float32),
            pltpu.SemaphoreType.DMA,
            pltpu.SemaphoreType.DMA,
        ],
        compiler_params=pltpu.CompilerParams(
            use_tc_tiling_on_sc=True, needs_layout_passes=False
        ),
    )
    def tcall(wt_hbm, tail_hbm, inter_hbm, buf0, buf1, tbuf0, tbuf1, sem0, sem1):
        bufs = (buf0, buf1)
        tbufs = (tbuf0, tbuf1)
        sems = (sem0, sem1)
        wid = lax.axis_index("s") * 2 + lax.axis_index("c")
        start = wid * _BPW + jnp.minimum(wid, _EXTRA)
        nblk = _BPW + jnp.where(wid < _EXTRA, 1, 0)
        lanes64 = lax.iota(jnp.int32, 16) * 64

        def stage_start(tc, b):
            for tr in range(8):
                pltpu.async_copy(
                    wt_hbm.at[pl.ds(tr * 8, 8), pl.ds(tc * 128, 128)],
                    bufs[b].at[pl.ds(tr * 8, 8)],
                    sems[b],
                )

        def stage_wait(tc, b):
            for tr in range(8):
                pltpu.make_async_copy(
                    wt_hbm.at[pl.ds(tr * 8, 8), pl.ds(tc * 128, 128)],
                    bufs[b].at[pl.ds(tr * 8, 8)],
                    sems[b],
                ).wait()

        def shuffle(b):
            def fbody(f, carry):
                for jg in range(8):
                    vec = bufs[b][f, pl.ds(jg * 16, 16)]
                    plsc.store_scatter(
                        tbufs[b], [lanes64 + (jg * 16 * 64 + f)], vec
                    )
                return carry

            lax.fori_loop(0, 64, fbody, 0)

        stage_start(start, 0)

        def body(k, carry):
            b = jnp.remainder(k, 2)
            tc = start + k

            @pl.when(b == 0)
            def _():
                stage_wait(tc, 0)

                @pl.when(k + 1 < nblk)
                def _():
                    stage_start(tc + 1, 1)

                shuffle(0)
                pltpu.sync_copy(tbuf0, inter_hbm.at[pl.ds(tc * 8192, 8192)])

            @pl.when(b == 1)
            def _():
                stage_wait(tc, 1)

                @pl.when(k + 1 < nblk)
                def _():
                    stage_start(tc + 1, 0)

                shuffle(1)
                pltpu.sync_copy(tbuf1, inter_hbm.at[pl.ds(tc * 8192, 8192)])

            return carry

        lax.fori_loop(0, nblk, body, 0)

        @pl.when(wid == _NW - 1)
        def _():
            pltpu.sync_copy(tail_hbm, tbuf0.at[pl.ds(0, _TAIL * _D)])
            pltpu.sync_copy(
                tbuf0.at[pl.ds(0, _TAIL * _D)],
                inter_hbm.at[pl.ds(_FULL_BLOCKS * 128 * _D, _TAIL * _D)],
            )

    return tcall(wt, tail1d)


def _gather_call(idxf, inter):
    nch, ch = idxf.shape
    cpw = nch // _NW
    mesh = plsc.VectorSubcoreMesh(core_axis_name="c", subcore_axis_name="s")

    @functools.partial(
        pl.kernel,
        out_type=jax.ShapeDtypeStruct((nch * ch * _D,), jnp.float32),
        mesh=mesh,
        scratch_types=[
            pltpu.VMEM((cpw, ch), jnp.int32),
            pltpu.VMEM((ch, _D), jnp.float32),
            pltpu.VMEM((ch, _D), jnp.float32),
            pltpu.VMEM((8192,), jnp.float32),
            pltpu.SemaphoreType.DMA,
            pltpu.SemaphoreType.DMA,
        ],
        compiler_params=pltpu.CompilerParams(
            use_tc_tiling_on_sc=False, needs_layout_passes=False
        ),
    )
    def gcall(idx_hbm, inter_hbm, out_hbm, idx_v, buf0, buf1, tbuf, gs0, gs1):
        bufs = (buf0, buf1)
        gsems = (gs0, gs1)
        wid = lax.axis_index("s") * 2 + lax.axis_index("c")
        c0 = wid * cpw
        pltpu.sync_copy(idx_hbm.at[pl.ds(c0, cpw)], idx_v)
        lanes128 = lax.iota(jnp.int32, 16) * 128

        def g_start(k, b):
            pltpu.async_copy(inter_hbm.at[idx_v.at[k]], bufs[b], gsems[b])

        def g_wait(k, b):
            pltpu.make_async_copy(
                inter_hbm.at[idx_v.at[k]], bufs[b], gsems[b]
            ).wait()

        def process(k, b):
            def jbody(j, carry):
                for fg in range(4):
                    vec = bufs[b][j, pl.ds(fg * 16, 16)]
                    plsc.store_scatter(
                        tbuf, [lanes128 + (fg * 16 * 128 + j)], vec
                    )
                return carry

            lax.fori_loop(0, ch, jbody, 0)
            c = c0 + k
            f = c // 128
            tc = c - f * 128
            for tr in range(8):
                pltpu.sync_copy(
                    tbuf.at[pl.ds(tr * 1024, 1024)],
                    out_hbm.at[pl.ds(f * 1048576 + tr * 131072 + tc * 1024, 1024)],
                )

        g_start(0, 0)
        g_start(1, 1)

        def body(i, carry):
            for b in range(2):
                k = 2 * i + b
                g_wait(k, b)
                process(k, b)
                g_start(k + 2, b)
            return carry

        lax.fori_loop(0, cpw // 2 - 1, body, 0)
        for b in range(2):
            k = cpw - 2 + b
            g_wait(k, b)
            process(k, b)

    return gcall(idxf, inter)


@jax.jit
def _embed(x, weight):
    batch, fields = x.shape
    idxf = x.T.reshape(fields * (batch // 128), 128).astype(jnp.int32)
    tail1d = weight[_FULL_BLOCKS * 128 :].reshape(_TAIL * _D)
    inter = _transpose_call(weight.T, tail1d).reshape(_V, _D)
    out1d = _gather_call(idxf, inter)
    out5 = out1d.reshape(fields, 8, batch // 128, 8, 128)
    return out5.transpose(2, 4, 0, 1, 3).reshape(batch, fields, _D)


def kernel(x, weight):
    return _embed(x, weight)
